# Initial kernel scaffold; baseline (speedup 1.0000x reference)
import functools
import jax
import jax.numpy as jnp
from jax import lax
from jax.experimental import pallas as pl
from jax.experimental.pallas import tpu as pltpu
from jax.experimental.pallas import tpu_sc as plsc

N = 10000
E = 320000
EMB = 128
LD = 8
HID = 64
ED = 16

NC = 2
NS = 16
NW = NC * NS

BE = 2000


def _sc_gather(table, idx, chunk):
    V, D = table.shape
    B = idx.shape[0]
    per_w = B // NW
    n_chunks = per_w // chunk
    mesh = plsc.VectorSubcoreMesh(core_axis_name="c", subcore_axis_name="s")

    @functools.partial(
        pl.kernel, mesh=mesh,
        out_type=jax.ShapeDtypeStruct((B, D), jnp.float32),
        scratch_types=[
            pltpu.VMEM((chunk,), jnp.int32),
            pltpu.VMEM((chunk, D), jnp.float32),
            pltpu.SemaphoreType.DMA,
        ],
    )
    def k(table_hbm, idx_hbm, out_hbm, idx_v, rows_v, sem):
        wid = lax.axis_index("s") * NC + lax.axis_index("c")

        def body(j, carry):
            base = wid * per_w + j * chunk
            pltpu.sync_copy(idx_hbm.at[pl.ds(base, chunk)], idx_v)
            pltpu.async_copy(table_hbm.at[idx_v], rows_v, sem).wait()
            pltpu.sync_copy(rows_v, out_hbm.at[pl.ds(base, chunk)])
            return carry

        lax.fori_loop(0, n_chunks, body, 0)

    return k(table, idx)


def _sc_scatter_sum(rows, idx, chunk):
    Dm = rows.shape[1]
    per_w = E // NW
    n_chunks = per_w // chunk
    rows_per_sub = N // NS
    mesh = plsc.VectorSubcoreMesh(core_axis_name="c", subcore_axis_name="s")
    zeros = jnp.zeros((N, Dm), jnp.float32)

    @functools.partial(
        pl.kernel, mesh=mesh,
        out_type=jax.ShapeDtypeStruct((NC * N, Dm), jnp.float32),
        scratch_types=[
            pltpu.VMEM((chunk,), jnp.int32),
            pltpu.VMEM((chunk, Dm), jnp.float32),
            pltpu.VMEM_SHARED((N, Dm), jnp.float32),
        ],
    )
    def k(rows_hbm, idx_hbm, zeros_hbm, out_hbm, idx_v, rows_v, acc):
        cid = lax.axis_index("c")
        sid = lax.axis_index("s")
        wid = sid * NC + cid
        pltpu.sync_copy(zeros_hbm.at[pl.ds(sid * rows_per_sub, rows_per_sub)],
                        acc.at[pl.ds(sid * rows_per_sub, rows_per_sub)])
        plsc.subcore_barrier()

        def body(j, carry):
            base = wid * per_w + j * chunk
            pltpu.sync_copy(idx_hbm.at[pl.ds(base, chunk)], idx_v)
            pltpu.sync_copy(rows_hbm.at[pl.ds(base, chunk)], rows_v)
            pltpu.sync_copy(rows_v, acc.at[idx_v], add=True)
            return carry

        lax.fori_loop(0, n_chunks, body, 0)
        plsc.subcore_barrier()
        pltpu.sync_copy(acc.at[pl.ds(sid * rows_per_sub, rows_per_sub)],
                        out_hbm.at[pl.ds(cid * N + sid * rows_per_sub,
                                         rows_per_sub)])

    return k(rows, idx, zeros)


def _edge_body(ea_ref, x_ref, W1_ref, b1_ref, W2p_ref, b2r_ref, out_ref):
    ea = ea_ref[...]
    x = x_ref[...]
    h = jnp.maximum(
        jnp.dot(ea, W1_ref[...], preferred_element_type=jnp.float32)
        + b1_ref[...], 0.0)
    T = jnp.dot(x, W2p_ref[...], preferred_element_type=jnp.float32)
    xb2 = jnp.dot(x, b2r_ref[...], preferred_element_type=jnp.float32)
    cols = []
    for o in range(LD):
        part = T[:, o * HID:(o + 1) * HID] * h
        cols.append(jnp.sum(part, axis=1, keepdims=True))
    msgs = jnp.concatenate(cols, axis=1) + xb2
    nb = msgs.shape[0]
    ones = jnp.ones((nb, 1), jnp.float32)
    pad = jnp.zeros((nb, ED - LD - 1), jnp.float32)
    out_ref[...] = jnp.concatenate([msgs, ones, pad], axis=1)


def _latent_body(part_ref, static_ref, rootW_ref, rootb_ref, out_ref):
    p = part_ref[0:N, :] + part_ref[N:2 * N, :]
    s = p[:, 0:LD]
    cnt = p[:, LD:LD + 1]
    aggr = s / jnp.maximum(cnt, 1.0)
    out_ref[...] = aggr + jnp.dot(
        static_ref[...], rootW_ref[...],
        preferred_element_type=jnp.float32) + rootb_ref[...]


def _decoder_body(ls_ref, ld_ref, P1a_ref, P1b_ref, pb1_ref, P2_ref,
                  pb2_ref, out_ref):
    g = jnp.maximum(
        jnp.dot(ls_ref[...], P1a_ref[...], preferred_element_type=jnp.float32)
        + jnp.dot(ld_ref[...], P1b_ref[...], preferred_element_type=jnp.float32)
        + pb1_ref[...], 0.0)
    out_ref[...] = jnp.dot(
        g, P2_ref[...], preferred_element_type=jnp.float32) + pb2_ref[...]


def _edge_msgs(edge_attr, x_src, W1, b1, W2p, b2r):
    grid = (E // BE,)
    full = lambda shape: pl.BlockSpec(shape, lambda i: (0, 0))
    return pl.pallas_call(
        _edge_body,
        grid=grid,
        in_specs=[
            pl.BlockSpec((BE, ED), lambda i: (i, 0)),
            pl.BlockSpec((BE, EMB), lambda i: (i, 0)),
            full((ED, HID)),
            full((1, HID)),
            full((EMB, HID * LD)),
            full((EMB, LD)),
        ],
        out_specs=pl.BlockSpec((BE, ED), lambda i: (i, 0)),
        out_shape=jax.ShapeDtypeStruct((E, ED), jnp.float32),
    )(edge_attr, x_src, W1, b1, W2p, b2r)


def _latent(partials, static, rootW, rootb):
    return pl.pallas_call(
        _latent_body,
        out_shape=jax.ShapeDtypeStruct((N, LD), jnp.float32),
    )(partials, static, rootW, rootb)


def _decoder(lat_src, lat_dst, P1a, P1b, pb1, P2, pb2):
    grid = (E // BE,)
    full = lambda shape: pl.BlockSpec(shape, lambda i: (0, 0))
    return pl.pallas_call(
        _decoder_body,
        grid=grid,
        in_specs=[
            pl.BlockSpec((BE, LD), lambda i: (i, 0)),
            pl.BlockSpec((BE, LD), lambda i: (i, 0)),
            full((LD, HID)),
            full((LD, HID)),
            full((1, HID)),
            full((HID, ED)),
            full((1, ED)),
        ],
        out_specs=pl.BlockSpec((BE, ED), lambda i: (i, 0)),
        out_shape=jax.ShapeDtypeStruct((E, ED), jnp.float32),
    )(lat_src, lat_dst, P1a, P1b, pb1, P2, pb2)


def kernel(static_embeddings, edge_index, edge_attr, W1, b1, W2, b2,
           rootW, rootb, P1, pb1, P2, pb2):
    src = edge_index[0]
    dst = edge_index[1]
    W2p = W2.reshape(HID, EMB, LD).transpose(1, 2, 0).reshape(EMB, HID * LD)
    b2r = b2.reshape(EMB, LD)

    x_src = _sc_gather(static_embeddings, src, 400)
    msgs16 = _edge_msgs(edge_attr, x_src, W1, b1.reshape(1, HID), W2p, b2r)
    partials = _sc_scatter_sum(msgs16, dst, 400)
    latent = _latent(partials, static_embeddings, rootW, rootb.reshape(1, LD))
    lat_src = _sc_gather(latent, src, 1000)
    lat_dst = _sc_gather(latent, dst, 1000)
    return _decoder(lat_src, lat_dst, P1[:LD], P1[LD:],
                    pb1.reshape(1, HID), P2, pb2.reshape(1, ED))


# speedup vs baseline: 3.2137x; 3.2137x over previous
import functools
import jax
import jax.numpy as jnp
from jax import lax
from jax.experimental import pallas as pl
from jax.experimental.pallas import tpu as pltpu
from jax.experimental.pallas import tpu_sc as plsc

N = 10000
E = 320000
EMB = 128
LD = 8
HID = 64
ED = 16

NC = 2
NS = 16
NW = NC * NS

BE = 2000


def _sc_gather(table, idx, chunk):
    V, D = table.shape
    B = idx.shape[0]
    per_w = B // NW
    n_chunks = per_w // chunk
    mesh = plsc.VectorSubcoreMesh(core_axis_name="c", subcore_axis_name="s")

    @functools.partial(
        pl.kernel, mesh=mesh,
        out_type=jax.ShapeDtypeStruct((B, D), jnp.float32),
        scratch_types=[
            pltpu.VMEM((chunk,), jnp.int32),
            pltpu.VMEM((chunk, D), jnp.float32),
            pltpu.SemaphoreType.DMA,
        ],
    )
    def k(table_hbm, idx_hbm, out_hbm, idx_v, rows_v, sem):
        wid = lax.axis_index("s") * NC + lax.axis_index("c")

        def body(j, carry):
            base = wid * per_w + j * chunk
            pltpu.sync_copy(idx_hbm.at[pl.ds(base, chunk)], idx_v)
            pltpu.async_copy(table_hbm.at[idx_v], rows_v, sem).wait()
            pltpu.sync_copy(rows_v, out_hbm.at[pl.ds(base, chunk)])
            return carry

        lax.fori_loop(0, n_chunks, body, 0)

    return k(table, idx)


NP = 10240
HALF = NP // NC
ACCR = HALF + 128


def _sc_scatter_sum(rows, idx2, chunk):
    Dm = rows.shape[1]
    per_w = E // NS
    n_chunks = per_w // chunk
    rows_per_sub = HALF // NS
    zrows_per_sub = ACCR // NS
    mesh = plsc.VectorSubcoreMesh(core_axis_name="c", subcore_axis_name="s")
    zeros = jnp.zeros((ACCR, Dm), jnp.float32)

    @functools.partial(
        pl.kernel, mesh=mesh,
        out_type=jax.ShapeDtypeStruct((NP, Dm), jnp.float32),
        scratch_types=[
            pltpu.VMEM((chunk,), jnp.int32),
            pltpu.VMEM((chunk, Dm), jnp.float32),
            pltpu.VMEM_SHARED((ACCR, Dm), jnp.float32),
        ],
    )
    def k(rows_hbm, idx_hbm, zeros_hbm, out_hbm, idx_v, rows_v, acc):
        cid = lax.axis_index("c")
        sid = lax.axis_index("s")
        lo = cid * HALF
        pltpu.sync_copy(zeros_hbm.at[pl.ds(sid * zrows_per_sub,
                                           zrows_per_sub)],
                        rows_v.at[pl.ds(0, zrows_per_sub)])
        pltpu.sync_copy(rows_v.at[pl.ds(0, zrows_per_sub)],
                        acc.at[pl.ds(sid * zrows_per_sub, zrows_per_sub)])
        plsc.subcore_barrier()

        def body(j, carry):
            base = sid * per_w + j * chunk
            pltpu.sync_copy(idx_hbm.at[pl.ds(cid * E + base, chunk)], idx_v)
            pltpu.sync_copy(rows_hbm.at[pl.ds(base, chunk)], rows_v)
            pltpu.sync_copy(rows_v, acc.at[idx_v], add=True)
            return carry

        lax.fori_loop(0, n_chunks, body, 0)
        plsc.subcore_barrier()
        pltpu.sync_copy(acc.at[pl.ds(sid * rows_per_sub, rows_per_sub)],
                        rows_v.at[pl.ds(0, rows_per_sub)])
        pltpu.sync_copy(rows_v.at[pl.ds(0, rows_per_sub)],
                        out_hbm.at[pl.ds(lo + sid * rows_per_sub,
                                         rows_per_sub)])

    return k(rows, idx2, zeros)


def _edge_body(ea_ref, x_ref, W1_ref, b1_ref, W2p_ref, R_ref, S16_ref,
               b2r16_ref, cr_ref, out_ref):
    ea = ea_ref[...]
    x = x_ref[...]
    h = jnp.maximum(
        jnp.dot(ea, W1_ref[...], preferred_element_type=jnp.float32)
        + b1_ref[...], 0.0)
    T = jnp.dot(x, W2p_ref[...], preferred_element_type=jnp.float32)
    ht = jnp.dot(h, R_ref[...], preferred_element_type=jnp.float32)
    U = T * ht
    out_ref[...] = (
        jnp.dot(U, S16_ref[...], preferred_element_type=jnp.float32)
        + jnp.dot(x, b2r16_ref[...], preferred_element_type=jnp.float32)
        + cr_ref[...])


def _latent_body(part_ref, static_ref, rootW_ref, rootb_ref,
                 P1a_ref, P1b_ref, pb1_ref, out_ref):
    p = part_ref[0:N, :]
    s = p[:, 0:LD]
    cnt = p[:, LD:LD + 1]
    aggr = s / jnp.maximum(cnt, 1.0)
    latent = aggr + jnp.dot(
        static_ref[...], rootW_ref[...],
        preferred_element_type=jnp.float32) + rootb_ref[...]
    left = jnp.dot(latent, P1a_ref[...],
                   preferred_element_type=jnp.float32) + pb1_ref[...]
    right = jnp.dot(latent, P1b_ref[...], preferred_element_type=jnp.float32)
    out_ref[...] = jnp.concatenate([left, right], axis=1)


def _decoder_body(cs_ref, cd_ref, P2_ref, pb2_ref, out_ref):
    g = jnp.maximum(cs_ref[:, 0:HID] + cd_ref[:, HID:EMB], 0.0)
    out_ref[...] = jnp.dot(
        g, P2_ref[...], preferred_element_type=jnp.float32) + pb2_ref[...]


def _edge_msgs(edge_attr, x_src, W1, b1, W2p, R, S16, b2r16, cr):
    grid = (E // BE,)
    full = lambda shape: pl.BlockSpec(shape, lambda i: (0, 0))
    return pl.pallas_call(
        _edge_body,
        grid=grid,
        in_specs=[
            pl.BlockSpec((BE, ED), lambda i: (i, 0)),
            pl.BlockSpec((BE, EMB), lambda i: (i, 0)),
            full((ED, HID)),
            full((1, HID)),
            full((EMB, HID * LD)),
            full((HID, HID * LD)),
            full((HID * LD, EMB)),
            full((EMB, EMB)),
            full((1, EMB)),
        ],
        out_specs=pl.BlockSpec((BE, EMB), lambda i: (i, 0)),
        out_shape=jax.ShapeDtypeStruct((E, EMB), jnp.float32),
    )(edge_attr, x_src, W1, b1, W2p, R, S16, b2r16, cr)


def _latent(partials, static, rootW, rootb, P1a, P1b, pb1):
    return pl.pallas_call(
        _latent_body,
        out_shape=jax.ShapeDtypeStruct((N, EMB), jnp.float32),
    )(partials, static, rootW, rootb, P1a, P1b, pb1)


def _decoder(cs, cd, P2, pb2):
    grid = (E // BE,)
    full = lambda shape: pl.BlockSpec(shape, lambda i: (0, 0))
    return pl.pallas_call(
        _decoder_body,
        grid=grid,
        in_specs=[
            pl.BlockSpec((BE, EMB), lambda i: (i, 0)),
            pl.BlockSpec((BE, EMB), lambda i: (i, 0)),
            full((HID, ED)),
            full((1, ED)),
        ],
        out_specs=pl.BlockSpec((BE, ED), lambda i: (i, 0)),
        out_shape=jax.ShapeDtypeStruct((E, ED), jnp.float32),
    )(cs, cd, P2, pb2)


def kernel(static_embeddings, edge_index, edge_attr, W1, b1, W2, b2,
           rootW, rootb, P1, pb1, P2, pb2):
    src = edge_index[0]
    dst = edge_index[1]
    W2p = W2.reshape(HID, EMB, LD).transpose(1, 2, 0).reshape(EMB, HID * LD)
    R = jnp.tile(jnp.eye(HID, dtype=jnp.float32), (1, LD))
    S16 = jnp.zeros((HID * LD, EMB), jnp.float32).at[
        jnp.arange(HID * LD), jnp.repeat(jnp.arange(LD), HID)].set(1.0)
    b2r16 = jnp.concatenate(
        [b2.reshape(EMB, LD), jnp.zeros((EMB, EMB - LD), jnp.float32)], axis=1)
    cr = jnp.zeros((1, EMB), jnp.float32).at[0, LD].set(1.0)

    x_src = _sc_gather(static_embeddings, src, 400)
    msgs16 = _edge_msgs(edge_attr, x_src, W1, b1.reshape(1, HID), W2p,
                        R, S16, b2r16, cr)
    rel0 = jnp.where(dst < HALF, dst, HALF)
    rel1 = jnp.where(dst >= HALF, dst - HALF, HALF)
    idx2 = jnp.concatenate([rel0, rel1])
    partials = _sc_scatter_sum(msgs16, idx2, 400)
    C = _latent(partials, static_embeddings, rootW, rootb.reshape(1, LD),
                P1[0:LD], P1[LD:2 * LD], pb1.reshape(1, HID))
    cs = _sc_gather(C, src, 400)
    cd = _sc_gather(C, dst, 400)
    return _decoder(cs, cd, P2, pb2.reshape(1, ED))
